# baseline (speedup 1.0000x reference)
import math

import jax
import jax.numpy as jnp
from jax import lax
from jax.experimental import pallas as pl
from jax.experimental.pallas import tpu as pltpu
from jax.experimental.pallas import tpu_sc as plsc

EMB = 64
SCALE = math.sqrt(EMB)
B_TOK = 4096 * 50
NC, NS, L = 2, 16, 16
NW = NC * NS
N_PER_W = B_TOK // NW
G = 128
K = 5
CHUNK = K * G
NCH = N_PER_W // CHUNK
IDX_ROWS = N_PER_W // G


def _emb_body(tok_hbm, table_hbm, out_hbm, idx_v, buf, gsem):
    wid = lax.axis_index("s") * NC + lax.axis_index("c")
    base = wid * N_PER_W
    pltpu.sync_copy(tok_hbm.at[wid], idx_v)

    def chunk_body(g, carry):
        cps = [
            pltpu.async_copy(
                table_hbm.at[idx_v.at[g * K + j]],
                buf.at[pl.ds(j * G, G)],
                gsem,
            )
            for j in range(K)
        ]
        for cp in cps:
            cp.wait()

        def mul_body(r, c):
            for l in range(EMB // L):
                buf[r, pl.ds(l * L, L)] = buf[r, pl.ds(l * L, L)] * SCALE
            return c

        lax.fori_loop(0, CHUNK, mul_body, 0, unroll=2)

        pltpu.sync_copy(buf, out_hbm.at[pl.ds(base + g * CHUNK, CHUNK)])
        return carry

    lax.fori_loop(0, NCH, chunk_body, 0)


@jax.jit
def _emb_call(tok3d, table):
    mesh = plsc.VectorSubcoreMesh(core_axis_name="c", subcore_axis_name="s")
    return pl.kernel(
        _emb_body,
        mesh=mesh,
        compiler_params=pltpu.CompilerParams(use_tc_tiling_on_sc=False),
        out_type=jax.ShapeDtypeStruct((B_TOK, EMB), jnp.float32),
        scratch_types=[
            pltpu.VMEM((IDX_ROWS, G), jnp.int32),
            pltpu.VMEM((CHUNK, EMB), jnp.float32),
            pltpu.SemaphoreType.DMA,
        ],
    )(tok3d, table)


def kernel(tokens, table):
    tok3d = tokens.astype(jnp.int32).reshape(NW, IDX_ROWS, G)
    out = _emb_call(tok3d, table)
    return out.reshape(tokens.shape[0], tokens.shape[1], EMB)
